# baseline (speedup 1.0000x reference)
import functools

import jax
import jax.numpy as jnp
from jax import lax
from jax.experimental import pallas as pl
from jax.experimental.pallas import tpu as pltpu
from jax.experimental.pallas import tpu_sc as plsc

B = 16384
EMB_D = 32
YEAR_DIM = 8
OUT_D = 64

NC, NS = 2, 16
NW = NC * NS
B_PER_W = B // NW
CHUNK = 128
NCHUNK = B_PER_W // CHUNK


def _gather_body(emb_m_hbm, emb_p_hbm, manu_hbm, part_hbm, m_out, p_out,
                 idx_m, idx_p, rows_m, rows_p, sem):
    wid = lax.axis_index("s") * NC + lax.axis_index("c")
    base = wid * B_PER_W
    for j in range(NCHUNK):
        pltpu.sync_copy(manu_hbm.at[pl.ds(base + j * CHUNK, CHUNK)], idx_m.at[j])
        pltpu.sync_copy(part_hbm.at[pl.ds(base + j * CHUNK, CHUNK)], idx_p.at[j])
    copies = []
    for j in range(NCHUNK):
        copies.append(pltpu.async_copy(
            emb_m_hbm.at[idx_m.at[j]], rows_m.at[pl.ds(j * CHUNK, CHUNK)], sem))
        copies.append(pltpu.async_copy(
            emb_p_hbm.at[idx_p.at[j]], rows_p.at[pl.ds(j * CHUNK, CHUNK)], sem))
    for c in copies:
        c.wait()
    pltpu.sync_copy(rows_m, m_out.at[pl.ds(base, B_PER_W)])
    pltpu.sync_copy(rows_p, p_out.at[pl.ds(base, B_PER_W)])


def _sc_gather(emb_manu, emb_part, manu, part):
    kfn = pl.kernel(
        _gather_body,
        out_type=[jax.ShapeDtypeStruct((B, EMB_D), jnp.float32),
                  jax.ShapeDtypeStruct((B, EMB_D), jnp.float32)],
        mesh=plsc.VectorSubcoreMesh(core_axis_name="c", subcore_axis_name="s"),
        scratch_types=[
            pltpu.VMEM((NCHUNK, CHUNK), jnp.int32),
            pltpu.VMEM((NCHUNK, CHUNK), jnp.int32),
            pltpu.VMEM((B_PER_W, EMB_D), jnp.float32),
            pltpu.VMEM((B_PER_W, EMB_D), jnp.float32),
            pltpu.SemaphoreType.DMA,
        ],
        compiler_params=pltpu.CompilerParams(use_tc_tiling_on_sc=False),
    )
    return kfn(emb_manu, emb_part, manu, part)


BLK = 2048


def _tc_body(year_ref, m_ref, p_ref, W1_ref, b1_ref, W2_ref, b2_ref,
             Wpy_ref, Wpm_ref, Wpp_ref, bp_ref, out_ref):
    y = jnp.maximum(year_ref[...] * W1_ref[...] + b1_ref[...], 0.0)
    y = jnp.maximum(
        jnp.dot(y, W2_ref[...], preferred_element_type=jnp.float32) + b2_ref[...],
        0.0)
    acc = jnp.dot(y, Wpy_ref[...], preferred_element_type=jnp.float32)
    acc += jnp.dot(m_ref[...], Wpm_ref[...], preferred_element_type=jnp.float32)
    acc += jnp.dot(p_ref[...], Wpp_ref[...], preferred_element_type=jnp.float32)
    out_ref[...] = jnp.maximum(acc + bp_ref[...], 0.0)


def _tc_dense(year, m, p, W1, b1, W2, b2, Wp, bp):
    Wpy = Wp[0:YEAR_DIM]
    Wpm = Wp[YEAR_DIM:YEAR_DIM + EMB_D]
    Wpp = Wp[YEAR_DIM + EMB_D:YEAR_DIM + 2 * EMB_D]
    rep = lambda shape: pl.BlockSpec(shape, lambda i: (0, 0))
    return pl.pallas_call(
        _tc_body,
        grid=(B // BLK,),
        in_specs=[
            pl.BlockSpec((BLK, 1), lambda i: (i, 0)),
            pl.BlockSpec((BLK, EMB_D), lambda i: (i, 0)),
            pl.BlockSpec((BLK, EMB_D), lambda i: (i, 0)),
            rep((1, YEAR_DIM)),
            rep((1, YEAR_DIM)),
            rep((YEAR_DIM, YEAR_DIM)),
            rep((1, YEAR_DIM)),
            rep((YEAR_DIM, OUT_D)),
            rep((EMB_D, OUT_D)),
            rep((EMB_D, OUT_D)),
            rep((1, OUT_D)),
        ],
        out_specs=pl.BlockSpec((BLK, OUT_D), lambda i: (i, 0)),
        out_shape=jax.ShapeDtypeStruct((B, OUT_D), jnp.float32),
    )(year, m, p, W1, b1.reshape(1, YEAR_DIM), W2, b2.reshape(1, YEAR_DIM),
      Wpy, Wpm, Wpp, bp.reshape(1, OUT_D))


def kernel(year, manu, part, emb_manu, emb_part, W1, b1, W2, b2, Wp, bp):
    manu = manu.astype(jnp.int32)
    part = part.astype(jnp.int32)
    m, p = _sc_gather(emb_manu, emb_part, manu, part)
    return _tc_dense(year, m, p, W1, b1, W2, b2, Wp, bp)


# speedup vs baseline: 1.4041x; 1.4041x over previous
import jax
import jax.numpy as jnp
from jax import lax
from jax.experimental import pallas as pl
from jax.experimental.pallas import tpu as pltpu
from jax.experimental.pallas import tpu_sc as plsc

B = 16384
V = 1000000
EMB_D = 32
YEAR_DIM = 8
OUT_D = 64

S = 250880
GBLK = 1024
NG = S // GBLK
TAIL_START = 3 * S
TAIL_W = V - TAIL_START

NC, NS = 2, 16
NW = NC * NS
B_PER_W = B // NW
CH = 128
NCHUNK = B_PER_W // CH


def _repack_body(m0, m1, m2, mt, p0, p1, p2, pt, wm, wp):
    wm[...] = jnp.concatenate(
        [m0[...].T, m1[...].T, m2[...].T, mt[...].T], axis=1)
    wp[...] = jnp.concatenate(
        [p0[...].T, p1[...].T, p2[...].T, pt[...].T], axis=1)


def _tc_repack(embT_m, tail_m, embT_p, tail_p):
    def specs():
        for c in range(3):
            yield pl.BlockSpec((EMB_D, GBLK), lambda i, c=c: (0, c * NG + i))
        yield pl.BlockSpec((EMB_D, GBLK), lambda i: (0, i))

    return pl.pallas_call(
        _repack_body,
        grid=(NG,),
        in_specs=[*specs(), *specs()],
        out_specs=[pl.BlockSpec((GBLK, 128), lambda i: (i, 0))] * 2,
        out_shape=[jax.ShapeDtypeStruct((S, 128), jnp.float32)] * 2,
    )(embT_m, embT_m, embT_m, tail_m, embT_p, embT_p, embT_p, tail_p)


def _gather_body(wide_m, wide_p, gm_hbm, gp_hbm, mW_out, pW_out,
                 idxv, rows, sem):
    wid = lax.axis_index("s") * NC + lax.axis_index("c")
    base = pl.multiple_of(wid * B_PER_W, B_PER_W)

    def one_table(wide, g_hbm, out):
        for j in range(NCHUNK):
            pltpu.sync_copy(g_hbm.at[pl.ds(base + j * CH, CH)], idxv.at[j])
        for j in range(NCHUNK):
            pltpu.async_copy(wide.at[idxv.at[j]],
                             rows.at[pl.ds(j * CH, CH)], sem)
        for j in range(NCHUNK):
            pltpu.make_async_copy(wide.at[pl.ds(0, CH)],
                                  rows.at[pl.ds(j * CH, CH)], sem).wait()
        pltpu.sync_copy(rows, out.at[pl.ds(base, B_PER_W)])

    one_table(wide_m, gm_hbm, mW_out)
    one_table(wide_p, gp_hbm, pW_out)


def _sc_gather(wide_m, wide_p, gm, gp):
    kfn = pl.kernel(
        _gather_body,
        out_type=[jax.ShapeDtypeStruct((B, 128), jnp.float32),
                  jax.ShapeDtypeStruct((B, 128), jnp.float32)],
        mesh=plsc.VectorSubcoreMesh(core_axis_name="c", subcore_axis_name="s"),
        scratch_types=[
            pltpu.VMEM((NCHUNK, CH), jnp.int32),
            pltpu.VMEM((B_PER_W, 128), jnp.float32),
            pltpu.SemaphoreType.DMA,
        ],
        compiler_params=pltpu.CompilerParams(use_tc_tiling_on_sc=True),
    )
    return kfn(wide_m, wide_p, gm, gp)


BLK = 2048


def _tc_body(year_ref, mW_ref, pW_ref, cm_ref, cp_ref, W1_ref, b1_ref,
             W2_ref, b2_ref, Wpy_ref, Wpm_ref, Wpp_ref, bp_ref, out_ref):
    def select(wide, c_col):
        acc = jnp.where(c_col == 0, wide[:, 0:EMB_D], 0.0)
        for c in range(1, 4):
            acc += jnp.where(c_col == c,
                             wide[:, c * EMB_D:(c + 1) * EMB_D], 0.0)
        return acc

    m = select(mW_ref[...], cm_ref[...])
    p = select(pW_ref[...], cp_ref[...])
    y = jnp.maximum(year_ref[...] * W1_ref[...] + b1_ref[...], 0.0)
    y = jnp.maximum(
        jnp.dot(y, W2_ref[...], preferred_element_type=jnp.float32) + b2_ref[...],
        0.0)
    acc = jnp.dot(y, Wpy_ref[...], preferred_element_type=jnp.float32)
    acc += jnp.dot(m, Wpm_ref[...], preferred_element_type=jnp.float32)
    acc += jnp.dot(p, Wpp_ref[...], preferred_element_type=jnp.float32)
    out_ref[...] = jnp.maximum(acc + bp_ref[...], 0.0)


def _tc_dense(year, mW, pW, cm, cp, W1, b1, W2, b2, Wp, bp):
    Wpy = Wp[0:YEAR_DIM]
    Wpm = Wp[YEAR_DIM:YEAR_DIM + EMB_D]
    Wpp = Wp[YEAR_DIM + EMB_D:YEAR_DIM + 2 * EMB_D]
    rep = lambda shape: pl.BlockSpec(shape, lambda i: (0, 0))
    return pl.pallas_call(
        _tc_body,
        grid=(B // BLK,),
        in_specs=[
            pl.BlockSpec((BLK, 1), lambda i: (i, 0)),
            pl.BlockSpec((BLK, 128), lambda i: (i, 0)),
            pl.BlockSpec((BLK, 128), lambda i: (i, 0)),
            pl.BlockSpec((BLK, 1), lambda i: (i, 0)),
            pl.BlockSpec((BLK, 1), lambda i: (i, 0)),
            rep((1, YEAR_DIM)),
            rep((1, YEAR_DIM)),
            rep((YEAR_DIM, YEAR_DIM)),
            rep((1, YEAR_DIM)),
            rep((YEAR_DIM, OUT_D)),
            rep((EMB_D, OUT_D)),
            rep((EMB_D, OUT_D)),
            rep((1, OUT_D)),
        ],
        out_specs=pl.BlockSpec((BLK, OUT_D), lambda i: (i, 0)),
        out_shape=jax.ShapeDtypeStruct((B, OUT_D), jnp.float32),
    )(year, mW, pW, cm, cp, W1, b1.reshape(1, YEAR_DIM), W2,
      b2.reshape(1, YEAR_DIM), Wpy, Wpm, Wpp, bp.reshape(1, OUT_D))


def kernel(year, manu, part, emb_manu, emb_part, W1, b1, W2, b2, Wp, bp):
    manu = manu.astype(jnp.int32)
    part = part.astype(jnp.int32)
    embT_m = emb_manu.T
    embT_p = emb_part.T
    pad = ((0, 0), (0, S - TAIL_W))
    tail_m = jnp.pad(embT_m[:, TAIL_START:], pad)
    tail_p = jnp.pad(embT_p[:, TAIL_START:], pad)
    wide_m, wide_p = _tc_repack(embT_m, tail_m, embT_p, tail_p)
    gm = manu % S
    gp = part % S
    mW, pW = _sc_gather(wide_m, wide_p, gm, gp)
    cm = (manu // S).reshape(B, 1)
    cp = (part // S).reshape(B, 1)
    return _tc_dense(year, mW, pW, cm, cp, W1, b1, W2, b2, Wp, bp)


# speedup vs baseline: 1.5196x; 1.0823x over previous
import jax
import jax.numpy as jnp
from jax import lax
from jax.experimental import pallas as pl
from jax.experimental.pallas import tpu as pltpu
from jax.experimental.pallas import tpu_sc as plsc

B = 16384
V = 1000000
EMB_D = 32
YEAR_DIM = 8
OUT_D = 64

S = 250880
GBLK = 1024
NG = S // GBLK
TAIL_START = 3 * S
TAIL_W = V - TAIL_START

NC, NS = 2, 16
NW = NC * NS
B_PER_W = B // NW
CH = 128
NCHUNK = B_PER_W // CH


def _repack_body(eye_ref, m0, m1, m2, mt, p0, p1, p2, pt, wm, wp):
    eye = eye_ref[...]
    t = lambda x: lax.dot_general(x[...], eye,
                                  dimension_numbers=(((0,), (0,)), ((), ())),
                                  preferred_element_type=jnp.float32)
    wm[...] = jnp.concatenate([t(m0), t(m1), t(m2), t(mt)], axis=1)
    wp[...] = jnp.concatenate([t(p0), t(p1), t(p2), t(pt)], axis=1)


def _tc_repack(embT_m, embT_p):
    def specs():
        for c in range(4):
            yield pl.BlockSpec((EMB_D, GBLK),
                               lambda i, c=c: (0, jnp.minimum(c * NG + i,
                                                              V // GBLK)))

    eye = jnp.eye(EMB_D, dtype=jnp.float32)
    return pl.pallas_call(
        _repack_body,
        grid=(NG,),
        in_specs=[pl.BlockSpec((EMB_D, EMB_D), lambda i: (0, 0)),
                  *specs(), *specs()],
        out_specs=[pl.BlockSpec((GBLK, 128), lambda i: (i, 0))] * 2,
        out_shape=[jax.ShapeDtypeStruct((S, 128), jnp.float32)] * 2,
    )(eye, embT_m, embT_m, embT_m, embT_m, embT_p, embT_p, embT_p, embT_p)


def _gather_body(wide_m, wide_p, gm_hbm, gp_hbm, mW_out, pW_out,
                 idxv, rows, sem):
    wid = lax.axis_index("s") * NC + lax.axis_index("c")
    base = pl.multiple_of(wid * B_PER_W, B_PER_W)

    def one_table(wide, g_hbm, out):
        for j in range(NCHUNK):
            pltpu.sync_copy(g_hbm.at[pl.ds(base + j * CH, CH)], idxv.at[j])
        for j in range(NCHUNK):
            pltpu.async_copy(wide.at[idxv.at[j]],
                             rows.at[pl.ds(j * CH, CH)], sem)
        for j in range(NCHUNK):
            pltpu.make_async_copy(wide.at[pl.ds(0, CH)],
                                  rows.at[pl.ds(j * CH, CH)], sem).wait()
        pltpu.sync_copy(rows, out.at[pl.ds(base, B_PER_W)])

    one_table(wide_m, gm_hbm, mW_out)
    one_table(wide_p, gp_hbm, pW_out)


def _sc_gather(wide_m, wide_p, gm, gp):
    kfn = pl.kernel(
        _gather_body,
        out_type=[jax.ShapeDtypeStruct((B, 128), jnp.float32),
                  jax.ShapeDtypeStruct((B, 128), jnp.float32)],
        mesh=plsc.VectorSubcoreMesh(core_axis_name="c", subcore_axis_name="s"),
        scratch_types=[
            pltpu.VMEM((NCHUNK, CH), jnp.int32),
            pltpu.VMEM((B_PER_W, 128), jnp.float32),
            pltpu.SemaphoreType.DMA,
        ],
        compiler_params=pltpu.CompilerParams(use_tc_tiling_on_sc=True),
    )
    return kfn(wide_m, wide_p, gm, gp)


BLK = 2048


def _tc_body(year_ref, mW_ref, pW_ref, cm_ref, cp_ref, W1_ref, b1_ref,
             W2_ref, b2_ref, Wpy_ref, Wpm_ref, Wpp_ref, bp_ref, out_ref):
    def select(wide, c_col):
        acc = jnp.where(c_col == 0, wide[:, 0:EMB_D], 0.0)
        for c in range(1, 4):
            acc += jnp.where(c_col == c,
                             wide[:, c * EMB_D:(c + 1) * EMB_D], 0.0)
        return acc

    m = select(mW_ref[...], cm_ref[...])
    p = select(pW_ref[...], cp_ref[...])
    y = jnp.maximum(year_ref[...] * W1_ref[...] + b1_ref[...], 0.0)
    y = jnp.maximum(
        jnp.dot(y, W2_ref[...], preferred_element_type=jnp.float32) + b2_ref[...],
        0.0)
    acc = jnp.dot(y, Wpy_ref[...], preferred_element_type=jnp.float32)
    acc += jnp.dot(m, Wpm_ref[...], preferred_element_type=jnp.float32)
    acc += jnp.dot(p, Wpp_ref[...], preferred_element_type=jnp.float32)
    out_ref[...] = jnp.maximum(acc + bp_ref[...], 0.0)


def _tc_dense(year, mW, pW, cm, cp, W1, b1, W2, b2, Wp, bp):
    Wpy = Wp[0:YEAR_DIM]
    Wpm = Wp[YEAR_DIM:YEAR_DIM + EMB_D]
    Wpp = Wp[YEAR_DIM + EMB_D:YEAR_DIM + 2 * EMB_D]
    rep = lambda shape: pl.BlockSpec(shape, lambda i: (0, 0))
    return pl.pallas_call(
        _tc_body,
        grid=(B // BLK,),
        in_specs=[
            pl.BlockSpec((BLK, 1), lambda i: (i, 0)),
            pl.BlockSpec((BLK, 128), lambda i: (i, 0)),
            pl.BlockSpec((BLK, 128), lambda i: (i, 0)),
            pl.BlockSpec((BLK, 1), lambda i: (i, 0)),
            pl.BlockSpec((BLK, 1), lambda i: (i, 0)),
            rep((1, YEAR_DIM)),
            rep((1, YEAR_DIM)),
            rep((YEAR_DIM, YEAR_DIM)),
            rep((1, YEAR_DIM)),
            rep((YEAR_DIM, OUT_D)),
            rep((EMB_D, OUT_D)),
            rep((EMB_D, OUT_D)),
            rep((1, OUT_D)),
        ],
        out_specs=pl.BlockSpec((BLK, OUT_D), lambda i: (i, 0)),
        out_shape=jax.ShapeDtypeStruct((B, OUT_D), jnp.float32),
    )(year, mW, pW, cm, cp, W1, b1.reshape(1, YEAR_DIM), W2,
      b2.reshape(1, YEAR_DIM), Wpy, Wpm, Wpp, bp.reshape(1, OUT_D))


def kernel(year, manu, part, emb_manu, emb_part, W1, b1, W2, b2, Wp, bp):
    manu = manu.astype(jnp.int32)
    part = part.astype(jnp.int32)
    wide_m, wide_p = _tc_repack(emb_manu.T, emb_part.T)
    gm = manu % S
    gp = part % S
    mW, pW = _sc_gather(wide_m, wide_p, gm, gp)
    cm = (manu // S).reshape(B, 1)
    cp = (part // S).reshape(B, 1)
    return _tc_dense(year, mW, pW, cm, cp, W1, b1, W2, b2, Wp, bp)


# speedup vs baseline: 1.6187x; 1.0652x over previous
import jax
import jax.numpy as jnp
from jax import lax
from jax.experimental import pallas as pl
from jax.experimental.pallas import tpu as pltpu
from jax.experimental.pallas import tpu_sc as plsc

B = 16384
V = 1000000
EMB_D = 32
YEAR_DIM = 8
OUT_D = 64

S = 253952
GBLK = 4096
NG = S // GBLK
TAIL_START = 3 * S
TAIL_W = V - TAIL_START

NC, NS = 2, 16
NW = NC * NS
B_PER_W = B // NW
CH = 128
NCHUNK = B_PER_W // CH


def _repack_body(eye_ref, m0, m1, m2, mt, p0, p1, p2, pt, wm, wp):
    eye = eye_ref[...]
    t = lambda x: lax.dot_general(x[...], eye,
                                  dimension_numbers=(((0,), (0,)), ((), ())),
                                  preferred_element_type=jnp.float32)
    wm[...] = jnp.concatenate([t(m0), t(m1), t(m2), t(mt)], axis=1)
    wp[...] = jnp.concatenate([t(p0), t(p1), t(p2), t(pt)], axis=1)


def _tc_repack(embT_m, embT_p):
    def specs():
        for c in range(4):
            yield pl.BlockSpec((EMB_D, GBLK),
                               lambda i, c=c: (0, jnp.minimum(c * NG + i,
                                                              V // GBLK)))

    eye = jnp.eye(EMB_D, dtype=jnp.float32)
    return pl.pallas_call(
        _repack_body,
        grid=(NG,),
        in_specs=[pl.BlockSpec((EMB_D, EMB_D), lambda i: (0, 0)),
                  *specs(), *specs()],
        out_specs=[pl.BlockSpec((GBLK, 128), lambda i: (i, 0))] * 2,
        out_shape=[jax.ShapeDtypeStruct((S, 128), jnp.float32)] * 2,
    )(eye, embT_m, embT_m, embT_m, embT_m, embT_p, embT_p, embT_p, embT_p)


def _gather_body(wide_m, wide_p, gm_hbm, gp_hbm, mW_out, pW_out,
                 idxv, rows, sem):
    wid = lax.axis_index("s") * NC + lax.axis_index("c")
    base = pl.multiple_of(wid * B_PER_W, B_PER_W)

    def one_table(wide, g_hbm, out):
        for j in range(NCHUNK):
            pltpu.sync_copy(g_hbm.at[pl.ds(base + j * CH, CH)], idxv.at[j])
        for j in range(NCHUNK):
            pltpu.async_copy(wide.at[idxv.at[j]],
                             rows.at[pl.ds(j * CH, CH)], sem)
        for j in range(NCHUNK):
            pltpu.make_async_copy(wide.at[pl.ds(0, CH)],
                                  rows.at[pl.ds(j * CH, CH)], sem).wait()
        pltpu.sync_copy(rows, out.at[pl.ds(base, B_PER_W)])

    one_table(wide_m, gm_hbm, mW_out)
    one_table(wide_p, gp_hbm, pW_out)


def _sc_gather(wide_m, wide_p, gm, gp):
    kfn = pl.kernel(
        _gather_body,
        out_type=[jax.ShapeDtypeStruct((B, 128), jnp.float32),
                  jax.ShapeDtypeStruct((B, 128), jnp.float32)],
        mesh=plsc.VectorSubcoreMesh(core_axis_name="c", subcore_axis_name="s"),
        scratch_types=[
            pltpu.VMEM((NCHUNK, CH), jnp.int32),
            pltpu.VMEM((B_PER_W, 128), jnp.float32),
            pltpu.SemaphoreType.DMA,
        ],
        compiler_params=pltpu.CompilerParams(use_tc_tiling_on_sc=True),
    )
    return kfn(wide_m, wide_p, gm, gp)


BLK = 2048


def _tc_body(year_ref, mW_ref, pW_ref, cm_ref, cp_ref, W1_ref, b1_ref,
             W2_ref, b2_ref, Wpy_ref, Wpm_ref, Wpp_ref, bp_ref, out_ref):
    def select(wide, c_col):
        acc = jnp.where(c_col == 0, wide[:, 0:EMB_D], 0.0)
        for c in range(1, 4):
            acc += jnp.where(c_col == c,
                             wide[:, c * EMB_D:(c + 1) * EMB_D], 0.0)
        return acc

    m = select(mW_ref[...], cm_ref[...])
    p = select(pW_ref[...], cp_ref[...])
    y = jnp.maximum(year_ref[...] * W1_ref[...] + b1_ref[...], 0.0)
    y = jnp.maximum(
        jnp.dot(y, W2_ref[...], preferred_element_type=jnp.float32) + b2_ref[...],
        0.0)
    acc = jnp.dot(y, Wpy_ref[...], preferred_element_type=jnp.float32)
    acc += jnp.dot(m, Wpm_ref[...], preferred_element_type=jnp.float32)
    acc += jnp.dot(p, Wpp_ref[...], preferred_element_type=jnp.float32)
    out_ref[...] = jnp.maximum(acc + bp_ref[...], 0.0)


def _tc_dense(year, mW, pW, cm, cp, W1, b1, W2, b2, Wp, bp):
    Wpy = Wp[0:YEAR_DIM]
    Wpm = Wp[YEAR_DIM:YEAR_DIM + EMB_D]
    Wpp = Wp[YEAR_DIM + EMB_D:YEAR_DIM + 2 * EMB_D]
    rep = lambda shape: pl.BlockSpec(shape, lambda i: (0, 0))
    return pl.pallas_call(
        _tc_body,
        grid=(B // BLK,),
        in_specs=[
            pl.BlockSpec((BLK, 1), lambda i: (i, 0)),
            pl.BlockSpec((BLK, 128), lambda i: (i, 0)),
            pl.BlockSpec((BLK, 128), lambda i: (i, 0)),
            pl.BlockSpec((BLK, 1), lambda i: (i, 0)),
            pl.BlockSpec((BLK, 1), lambda i: (i, 0)),
            rep((1, YEAR_DIM)),
            rep((1, YEAR_DIM)),
            rep((YEAR_DIM, YEAR_DIM)),
            rep((1, YEAR_DIM)),
            rep((YEAR_DIM, OUT_D)),
            rep((EMB_D, OUT_D)),
            rep((EMB_D, OUT_D)),
            rep((1, OUT_D)),
        ],
        out_specs=pl.BlockSpec((BLK, OUT_D), lambda i: (i, 0)),
        out_shape=jax.ShapeDtypeStruct((B, OUT_D), jnp.float32),
    )(year, mW, pW, cm, cp, W1, b1.reshape(1, YEAR_DIM), W2,
      b2.reshape(1, YEAR_DIM), Wpy, Wpm, Wpp, bp.reshape(1, OUT_D))


def kernel(year, manu, part, emb_manu, emb_part, W1, b1, W2, b2, Wp, bp):
    manu = manu.astype(jnp.int32)
    part = part.astype(jnp.int32)
    wide_m, wide_p = _tc_repack(emb_manu.T, emb_part.T)
    gm = manu % S
    gp = part % S
    mW, pW = _sc_gather(wide_m, wide_p, gm, gp)
    cm = (manu // S).reshape(B, 1)
    cp = (part // S).reshape(B, 1)
    return _tc_dense(year, mW, pW, cm, cp, W1, b1, W2, b2, Wp, bp)


# speedup vs baseline: 2.4345x; 1.5040x over previous
import jax
import jax.numpy as jnp
from jax import lax
from jax.experimental import pallas as pl
from jax.experimental.pallas import tpu as pltpu
from jax.experimental.pallas import tpu_sc as plsc

B = 16384
V = 1000000
EMB_D = 32
YEAR_DIM = 8
OUT_D = 64

S = 253952
GBLK = 4096
NG = S // GBLK
TAIL_START = 3 * S
TAIL_W = V - TAIL_START

NC, NS = 2, 16
NW = NC * NS
B_PER_W = B // NW
CH = 128
NCHUNK = B_PER_W // CH


def _repack_body(sel_ref, m0, m1, m2, mt, p0, p1, p2, pt, wm, wp):
    def w(blocks):
        acc = None
        for c, x in enumerate(blocks):
            e = sel_ref[c]
            t = lax.dot_general(x[...], e,
                                dimension_numbers=(((0,), (0,)), ((), ())),
                                preferred_element_type=jnp.float32)
            acc = t if acc is None else acc + t
        return acc

    wm[...] = w([m0, m1, m2, mt])
    wp[...] = w([p0, p1, p2, pt])


def _tc_repack(embT_m, embT_p):
    def specs():
        for c in range(4):
            yield pl.BlockSpec((EMB_D, GBLK),
                               lambda i, c=c: (0, jnp.minimum(c * NG + i,
                                                              V // GBLK)))

    sel = jnp.zeros((4, EMB_D, 128), jnp.float32)
    for c in range(4):
        sel = sel.at[c, :, c * EMB_D:(c + 1) * EMB_D].set(
            jnp.eye(EMB_D, dtype=jnp.float32))
    return pl.pallas_call(
        _repack_body,
        grid=(NG,),
        in_specs=[pl.BlockSpec((4, EMB_D, 128), lambda i: (0, 0, 0)),
                  *specs(), *specs()],
        out_specs=[pl.BlockSpec((GBLK, 128), lambda i: (i, 0))] * 2,
        out_shape=[jax.ShapeDtypeStruct((S, 128), jnp.float32)] * 2,
    )(sel, embT_m, embT_m, embT_m, embT_m, embT_p, embT_p, embT_p, embT_p)


def _gather_body(wide_m, wide_p, gm_hbm, gp_hbm, mW_out, pW_out,
                 idxv, rows, sem):
    wid = lax.axis_index("s") * NC + lax.axis_index("c")
    base = pl.multiple_of(wid * B_PER_W, B_PER_W)

    def one_table(wide, g_hbm, out):
        for j in range(NCHUNK):
            pltpu.sync_copy(g_hbm.at[pl.ds(base + j * CH, CH)], idxv.at[j])
        for j in range(NCHUNK):
            pltpu.async_copy(wide.at[idxv.at[j]],
                             rows.at[pl.ds(j * CH, CH)], sem)
        for j in range(NCHUNK):
            pltpu.make_async_copy(wide.at[pl.ds(0, CH)],
                                  rows.at[pl.ds(j * CH, CH)], sem).wait()
        pltpu.sync_copy(rows, out.at[pl.ds(base, B_PER_W)])

    one_table(wide_m, gm_hbm, mW_out)
    one_table(wide_p, gp_hbm, pW_out)


def _sc_gather(wide_m, wide_p, gm, gp):
    kfn = pl.kernel(
        _gather_body,
        out_type=[jax.ShapeDtypeStruct((B, 128), jnp.float32),
                  jax.ShapeDtypeStruct((B, 128), jnp.float32)],
        mesh=plsc.VectorSubcoreMesh(core_axis_name="c", subcore_axis_name="s"),
        scratch_types=[
            pltpu.VMEM((NCHUNK, CH), jnp.int32),
            pltpu.VMEM((B_PER_W, 128), jnp.float32),
            pltpu.SemaphoreType.DMA,
        ],
        compiler_params=pltpu.CompilerParams(use_tc_tiling_on_sc=True),
    )
    return kfn(wide_m, wide_p, gm, gp)


BLK = 2048


def _tc_body(year_ref, mW_ref, pW_ref, cm_ref, cp_ref, W1_ref, b1_ref,
             W2_ref, b2_ref, Wpy_ref, Wpm_ref, Wpp_ref, bp_ref, out_ref):
    def select(wide, c_col):
        acc = jnp.where(c_col == 0, wide[:, 0:EMB_D], 0.0)
        for c in range(1, 4):
            acc += jnp.where(c_col == c,
                             wide[:, c * EMB_D:(c + 1) * EMB_D], 0.0)
        return acc

    m = select(mW_ref[...], cm_ref[...])
    p = select(pW_ref[...], cp_ref[...])
    y = jnp.maximum(year_ref[...] * W1_ref[...] + b1_ref[...], 0.0)
    y = jnp.maximum(
        jnp.dot(y, W2_ref[...], preferred_element_type=jnp.float32) + b2_ref[...],
        0.0)
    acc = jnp.dot(y, Wpy_ref[...], preferred_element_type=jnp.float32)
    acc += jnp.dot(m, Wpm_ref[...], preferred_element_type=jnp.float32)
    acc += jnp.dot(p, Wpp_ref[...], preferred_element_type=jnp.float32)
    out_ref[...] = jnp.maximum(acc + bp_ref[...], 0.0)


def _tc_dense(year, mW, pW, cm, cp, W1, b1, W2, b2, Wp, bp):
    Wpy = Wp[0:YEAR_DIM]
    Wpm = Wp[YEAR_DIM:YEAR_DIM + EMB_D]
    Wpp = Wp[YEAR_DIM + EMB_D:YEAR_DIM + 2 * EMB_D]
    rep = lambda shape: pl.BlockSpec(shape, lambda i: (0, 0))
    return pl.pallas_call(
        _tc_body,
        grid=(B // BLK,),
        in_specs=[
            pl.BlockSpec((BLK, 1), lambda i: (i, 0)),
            pl.BlockSpec((BLK, 128), lambda i: (i, 0)),
            pl.BlockSpec((BLK, 128), lambda i: (i, 0)),
            pl.BlockSpec((BLK, 1), lambda i: (i, 0)),
            pl.BlockSpec((BLK, 1), lambda i: (i, 0)),
            rep((1, YEAR_DIM)),
            rep((1, YEAR_DIM)),
            rep((YEAR_DIM, YEAR_DIM)),
            rep((1, YEAR_DIM)),
            rep((YEAR_DIM, OUT_D)),
            rep((EMB_D, OUT_D)),
            rep((EMB_D, OUT_D)),
            rep((1, OUT_D)),
        ],
        out_specs=pl.BlockSpec((BLK, OUT_D), lambda i: (i, 0)),
        out_shape=jax.ShapeDtypeStruct((B, OUT_D), jnp.float32),
    )(year, mW, pW, cm, cp, W1, b1.reshape(1, YEAR_DIM), W2,
      b2.reshape(1, YEAR_DIM), Wpy, Wpm, Wpp, bp.reshape(1, OUT_D))


def kernel(year, manu, part, emb_manu, emb_part, W1, b1, W2, b2, Wp, bp):
    manu = manu.astype(jnp.int32)
    part = part.astype(jnp.int32)
    wide_m, wide_p = _tc_repack(emb_manu.T, emb_part.T)
    gm = manu % S
    gp = part % S
    mW, pW = _sc_gather(wide_m, wide_p, gm, gp)
    cm = (manu // S).reshape(B, 1)
    cp = (part // S).reshape(B, 1)
    return _tc_dense(year, mW, pW, cm, cp, W1, b1, W2, b2, Wp, bp)
